# baseline (speedup 1.0000x reference)
import functools

import jax
import jax.numpy as jnp
from jax import lax
from jax.experimental import pallas as pl
from jax.experimental.pallas import tpu as pltpu
from jax.experimental.pallas import tpu_sc as plsc

VOCAB = 1000000
DIM = 64
BATCH = 1024
SEQ = 200

PDIM = 128
SPLIT_A = 96
SPLIT_B = 104
NWORKERS = 32
CPW = BATCH // NWORKERS
NBUF = 4
LOOKAHEAD = 2

TBLOCK = 512
HPAD = ((VOCAB // 2 + TBLOCK - 1) // TBLOCK) * TBLOCK
REMAP_SUB = 2 * HPAD - 1


def _positional_encoding(max_len, dim):
    pos = jnp.arange(max_len, dtype=jnp.float32)[:, None]
    div = jnp.exp(jnp.arange(0, dim, 2, dtype=jnp.float32) * (-jnp.log(10000.0) / dim))
    pe = jnp.zeros((max_len, dim), dtype=jnp.float32)
    pe = pe.at[:, 0::2].set(jnp.sin(pos * div))
    pe = pe.at[:, 1::2].set(jnp.cos(pos * div))
    return pe


def _transpose_body(lo_ref, hi_ref, out_ref):
    out_ref[:, 0:DIM] = lo_ref[...].T
    out_ref[:, DIM:PDIM] = hi_ref[...].T


def _table_rowmajor(token_table):
    tbl_t = token_table.T
    nblk = HPAD // TBLOCK
    return pl.pallas_call(
        _transpose_body,
        grid=(nblk,),
        in_specs=[
            pl.BlockSpec((DIM, TBLOCK), lambda g: (0, g)),
            pl.BlockSpec((DIM, TBLOCK), lambda g: (0, g + nblk)),
        ],
        out_specs=pl.BlockSpec((TBLOCK, PDIM), lambda g: (g, 0)),
        out_shape=jax.ShapeDtypeStruct((HPAD, PDIM), jnp.float32),
    )(tbl_t, tbl_t)


def _sc_body(idx_hbm, tbl_hbm, pe_hbm, out_hbm, idx_v, rows_v, pe_v, gsem, wsem):
    wid = lax.axis_index("s") * 2 + lax.axis_index("c")
    base = wid * CPW

    pltpu.sync_copy(pe_hbm, pe_v)
    pltpu.sync_copy(idx_hbm.at[pl.ds(base * SEQ, CPW * SEQ)], idx_v)

    def remap_body(v, carry):
        sl = pl.ds(v * 16, 16)
        i2 = idx_v[sl] * 2
        idx_v[sl] = jnp.where(i2 < 2 * HPAD, i2, i2 - REMAP_SUB)
        return carry

    lax.fori_loop(0, CPW * SEQ // 16, remap_body, 0, unroll=4)

    def start_gather(i):
        b = i % NBUF
        c0 = pltpu.async_copy(
            tbl_hbm.at[idx_v.at[pl.ds(i * SEQ, SPLIT_A)]],
            rows_v.at[b, pl.ds(0, SPLIT_A)], gsem.at[b])
        c1 = pltpu.async_copy(
            tbl_hbm.at[idx_v.at[pl.ds(i * SEQ + SPLIT_A, SPLIT_B)]],
            rows_v.at[b, pl.ds(SPLIT_A, SPLIT_B)], gsem.at[b])
        return c0, c1

    def start_writeback(i):
        b = i % NBUF
        return pltpu.async_copy(
            rows_v.at[b], out_hbm.at[pl.ds((base + i) * SEQ, SEQ)], wsem.at[b])

    gathers = {}
    wbs = {}
    for i in range(LOOKAHEAD):
        gathers[i] = start_gather(i)

    for i in range(CPW):
        b = i % NBUF
        g0, g1 = gathers.pop(i)
        g0.wait()
        g1.wait()

        def add_body(r, carry, _b=b):
            for k in range(DIM // 16):
                sl = pl.ds(k * 16, 16)
                rows_v[_b, r, sl] = rows_v[_b, r, sl] + pe_v[r, sl]
            return carry

        lax.fori_loop(0, SEQ, add_body, 0, unroll=2)
        wbs[i] = start_writeback(i)

        j = i + LOOKAHEAD
        if j < CPW:
            if j >= NBUF:
                wbs.pop(j - NBUF).wait()
            gathers[j] = start_gather(j)

    for i in sorted(wbs):
        wbs[i].wait()


def kernel(X, token_table):
    idx = X.astype(jnp.int32).reshape(BATCH * SEQ)
    pe = _positional_encoding(SEQ, DIM)
    tbl = _table_rowmajor(token_table).reshape(2 * HPAD, DIM)

    mesh = plsc.VectorSubcoreMesh(core_axis_name="c", subcore_axis_name="s")
    run = functools.partial(
        pl.kernel,
        mesh=mesh,
        compiler_params=pltpu.CompilerParams(use_tc_tiling_on_sc=False),
        out_type=jax.ShapeDtypeStruct((BATCH * SEQ, DIM), jnp.float32),
        scratch_types=[
            pltpu.VMEM((CPW * SEQ,), jnp.int32),
            pltpu.VMEM((NBUF, SEQ, DIM), jnp.float32),
            pltpu.VMEM((SEQ, DIM), jnp.float32),
            pltpu.SemaphoreType.DMA((NBUF,)),
            pltpu.SemaphoreType.DMA((NBUF,)),
        ],
    )(_sc_body)
    return run(idx, tbl, pe).reshape(BATCH, SEQ, DIM)


# speedup vs baseline: 1.3226x; 1.3226x over previous
import functools

import jax
import jax.numpy as jnp
from jax import lax
from jax.experimental import pallas as pl
from jax.experimental.pallas import tpu as pltpu
from jax.experimental.pallas import tpu_sc as plsc

VOCAB = 1000000
DIM = 64
BATCH = 1024
SEQ = 200

PDIM = 128
SPLIT_A = 96
SPLIT_B = 104
NWORKERS = 32
CPW = BATCH // NWORKERS
NBUF = 4
LOOKAHEAD = 2

TBLOCK = 2048


def _positional_encoding(max_len, dim):
    pos = jnp.arange(max_len, dtype=jnp.float32)[:, None]
    div = jnp.exp(jnp.arange(0, dim, 2, dtype=jnp.float32) * (-jnp.log(10000.0) / dim))
    pe = jnp.zeros((max_len, dim), dtype=jnp.float32)
    pe = pe.at[:, 0::2].set(jnp.sin(pos * div))
    pe = pe.at[:, 1::2].set(jnp.cos(pos * div))
    return pe


def _transpose_body(tbl_t_ref, out_ref):
    out_ref[:, 0:DIM] = tbl_t_ref[...].T


def _table_rowmajor(token_table):
    grid = (VOCAB + TBLOCK - 1) // TBLOCK
    return pl.pallas_call(
        _transpose_body,
        grid=(grid,),
        in_specs=[pl.BlockSpec((DIM, TBLOCK), lambda g: (0, g))],
        out_specs=pl.BlockSpec((TBLOCK, PDIM), lambda g: (g, 0)),
        out_shape=jax.ShapeDtypeStruct((VOCAB, PDIM), jnp.float32),
    )(token_table.T)


def _sc_body(idx_hbm, tbl_hbm, pe_hbm, out_hbm, idx_v, rows_v, pe_v, gsem, wsem):
    wid = lax.axis_index("s") * 2 + lax.axis_index("c")
    base = wid * CPW

    pltpu.sync_copy(pe_hbm, pe_v)
    pltpu.sync_copy(idx_hbm.at[pl.ds(base * SEQ, CPW * SEQ)], idx_v)

    def remap_body(v, carry):
        sl = pl.ds(v * 16, 16)
        idx_v[sl] = idx_v[sl] * 2
        return carry

    lax.fori_loop(0, CPW * SEQ // 16, remap_body, 0, unroll=4)

    def start_gather(i):
        b = i % NBUF
        c0 = pltpu.async_copy(
            tbl_hbm.at[idx_v.at[pl.ds(i * SEQ, SPLIT_A)]],
            rows_v.at[b, pl.ds(0, SPLIT_A)], gsem.at[b])
        c1 = pltpu.async_copy(
            tbl_hbm.at[idx_v.at[pl.ds(i * SEQ + SPLIT_A, SPLIT_B)]],
            rows_v.at[b, pl.ds(SPLIT_A, SPLIT_B)], gsem.at[b])
        return c0, c1

    def start_writeback(i):
        b = i % NBUF
        return pltpu.async_copy(
            rows_v.at[b], out_hbm.at[pl.ds((base + i) * SEQ, SEQ)], wsem.at[b])

    gathers = {}
    wbs = {}
    for i in range(LOOKAHEAD):
        gathers[i] = start_gather(i)

    for i in range(CPW):
        b = i % NBUF
        g0, g1 = gathers.pop(i)
        g0.wait()
        g1.wait()

        def add_body(r, carry, _b=b):
            for k in range(DIM // 16):
                sl = pl.ds(k * 16, 16)
                rows_v[_b, r, sl] = rows_v[_b, r, sl] + pe_v[r, sl]
            return carry

        lax.fori_loop(0, SEQ, add_body, 0, unroll=2)
        wbs[i] = start_writeback(i)

        j = i + LOOKAHEAD
        if j < CPW:
            if j >= NBUF:
                wbs.pop(j - NBUF).wait()
            gathers[j] = start_gather(j)

    for i in sorted(wbs):
        wbs[i].wait()


def kernel(X, token_table):
    idx = X.astype(jnp.int32).reshape(BATCH * SEQ)
    pe = _positional_encoding(SEQ, DIM)
    tbl = _table_rowmajor(token_table).reshape(2 * VOCAB, DIM)

    mesh = plsc.VectorSubcoreMesh(core_axis_name="c", subcore_axis_name="s")
    run = functools.partial(
        pl.kernel,
        mesh=mesh,
        compiler_params=pltpu.CompilerParams(use_tc_tiling_on_sc=False),
        out_type=jax.ShapeDtypeStruct((BATCH * SEQ, DIM), jnp.float32),
        scratch_types=[
            pltpu.VMEM((CPW * SEQ,), jnp.int32),
            pltpu.VMEM((NBUF, SEQ, DIM), jnp.float32),
            pltpu.VMEM((SEQ, DIM), jnp.float32),
            pltpu.SemaphoreType.DMA((NBUF,)),
            pltpu.SemaphoreType.DMA((NBUF,)),
        ],
    )(_sc_body)
    return run(idx, tbl, pe).reshape(BATCH, SEQ, DIM)


# speedup vs baseline: 1.8479x; 1.3972x over previous
import functools

import jax
import jax.numpy as jnp
from jax import lax
from jax.experimental import pallas as pl
from jax.experimental.pallas import tpu as pltpu
from jax.experimental.pallas import tpu_sc as plsc

VOCAB = 1000000
DIM = 64
BATCH = 1024
SEQ = 200

PDIM = 128
SPLIT_A = 96
SPLIT_B = 104
NWORKERS = 32
CPW = BATCH // NWORKERS
NBUF = 4
LOOKAHEAD = 2

TBLOCK = 8192


def _positional_encoding(max_len, dim):
    pos = jnp.arange(max_len, dtype=jnp.float32)[:, None]
    div = jnp.exp(jnp.arange(0, dim, 2, dtype=jnp.float32) * (-jnp.log(10000.0) / dim))
    pe = jnp.zeros((max_len, dim), dtype=jnp.float32)
    pe = pe.at[:, 0::2].set(jnp.sin(pos * div))
    pe = pe.at[:, 1::2].set(jnp.cos(pos * div))
    return pe


def _transpose_body(tbl_t_ref, out_ref):
    out_ref[:, 0:DIM] = tbl_t_ref[...].T


def _table_rowmajor(token_table):
    grid = (VOCAB + TBLOCK - 1) // TBLOCK
    return pl.pallas_call(
        _transpose_body,
        grid=(grid,),
        in_specs=[pl.BlockSpec((DIM, TBLOCK), lambda g: (0, g))],
        out_specs=pl.BlockSpec((TBLOCK, PDIM), lambda g: (g, 0)),
        out_shape=jax.ShapeDtypeStruct((VOCAB, PDIM), jnp.float32),
    )(token_table.T)


def _sc_body(idx_hbm, tbl_hbm, pe_hbm, out_hbm, idx_v, rows_v, pe_v, gsem, wsem):
    wid = lax.axis_index("s") * 2 + lax.axis_index("c")
    base = wid * CPW

    pltpu.sync_copy(pe_hbm, pe_v)
    pltpu.sync_copy(idx_hbm.at[pl.ds(base * SEQ, CPW * SEQ)], idx_v)

    def remap_body(v, carry):
        sl = pl.ds(v * 16, 16)
        idx_v[sl] = idx_v[sl] * 2
        return carry

    lax.fori_loop(0, CPW * SEQ // 16, remap_body, 0, unroll=4)

    def start_gather(i):
        b = i % NBUF
        c0 = pltpu.async_copy(
            tbl_hbm.at[idx_v.at[pl.ds(i * SEQ, SPLIT_A)]],
            rows_v.at[b, pl.ds(0, SPLIT_A)], gsem.at[b])
        c1 = pltpu.async_copy(
            tbl_hbm.at[idx_v.at[pl.ds(i * SEQ + SPLIT_A, SPLIT_B)]],
            rows_v.at[b, pl.ds(SPLIT_A, SPLIT_B)], gsem.at[b])
        return c0, c1

    def start_writeback(i):
        b = i % NBUF
        return pltpu.async_copy(
            rows_v.at[b], out_hbm.at[pl.ds((base + i) * SEQ, SEQ)], wsem.at[b])

    gathers = {}
    wbs = {}
    for i in range(LOOKAHEAD):
        gathers[i] = start_gather(i)

    for i in range(CPW):
        b = i % NBUF
        g0, g1 = gathers.pop(i)
        g0.wait()
        g1.wait()

        def add_body(r, carry, _b=b):
            for k in range(DIM // 16):
                sl = pl.ds(k * 16, 16)
                rows_v[_b, r, sl] = rows_v[_b, r, sl] + pe_v[r, sl]
            return carry

        lax.fori_loop(0, SEQ, add_body, 0, unroll=2)
        wbs[i] = start_writeback(i)

        j = i + LOOKAHEAD
        if j < CPW:
            if j >= NBUF:
                wbs.pop(j - NBUF).wait()
            gathers[j] = start_gather(j)

    for i in sorted(wbs):
        wbs[i].wait()


def kernel(X, token_table):
    idx = X.astype(jnp.int32).reshape(BATCH * SEQ)
    pe = _positional_encoding(SEQ, DIM)
    tbl = _table_rowmajor(token_table).reshape(2 * VOCAB, DIM)

    mesh = plsc.VectorSubcoreMesh(core_axis_name="c", subcore_axis_name="s")
    run = functools.partial(
        pl.kernel,
        mesh=mesh,
        compiler_params=pltpu.CompilerParams(use_tc_tiling_on_sc=False),
        out_type=jax.ShapeDtypeStruct((BATCH * SEQ, DIM), jnp.float32),
        scratch_types=[
            pltpu.VMEM((CPW * SEQ,), jnp.int32),
            pltpu.VMEM((NBUF, SEQ, DIM), jnp.float32),
            pltpu.VMEM((SEQ, DIM), jnp.float32),
            pltpu.SemaphoreType.DMA((NBUF,)),
            pltpu.SemaphoreType.DMA((NBUF,)),
        ],
    )(_sc_body)
    return run(idx, tbl, pe).reshape(BATCH, SEQ, DIM)


# speedup vs baseline: 1.9250x; 1.0417x over previous
import functools

import jax
import jax.numpy as jnp
from jax import lax
from jax.experimental import pallas as pl
from jax.experimental.pallas import tpu as pltpu
from jax.experimental.pallas import tpu_sc as plsc

VOCAB = 1000000
DIM = 64
BATCH = 1024
SEQ = 200

PDIM = 128
SPLIT_A = 96
SPLIT_B = 104
NWORKERS = 32
CPW = BATCH // NWORKERS
NBUF = 4
LOOKAHEAD = 2

TBLOCK = 16384


def _positional_encoding(max_len, dim):
    pos = jnp.arange(max_len, dtype=jnp.float32)[:, None]
    div = jnp.exp(jnp.arange(0, dim, 2, dtype=jnp.float32) * (-jnp.log(10000.0) / dim))
    pe = jnp.zeros((max_len, dim), dtype=jnp.float32)
    pe = pe.at[:, 0::2].set(jnp.sin(pos * div))
    pe = pe.at[:, 1::2].set(jnp.cos(pos * div))
    return pe


def _transpose_body(tbl_t_ref, out_ref):
    out_ref[:, 0:DIM] = tbl_t_ref[...].T


def _table_rowmajor(token_table):
    grid = (VOCAB + TBLOCK - 1) // TBLOCK
    return pl.pallas_call(
        _transpose_body,
        grid=(grid,),
        in_specs=[pl.BlockSpec((DIM, TBLOCK), lambda g: (0, g))],
        out_specs=pl.BlockSpec((TBLOCK, PDIM), lambda g: (g, 0)),
        out_shape=jax.ShapeDtypeStruct((VOCAB, PDIM), jnp.float32),
    )(token_table.T)


def _sc_body(idx_hbm, tbl_hbm, pe_hbm, out_hbm, idx_v, rows_v, pe_v, gsem, wsem):
    wid = lax.axis_index("s") * 2 + lax.axis_index("c")
    base = wid * CPW

    pltpu.sync_copy(pe_hbm, pe_v)
    pltpu.sync_copy(idx_hbm.at[pl.ds(base * SEQ, CPW * SEQ)], idx_v)

    def remap_body(v, carry):
        sl = pl.ds(v * 16, 16)
        idx_v[sl] = idx_v[sl] * 2
        return carry

    lax.fori_loop(0, CPW * SEQ // 16, remap_body, 0, unroll=4)

    def start_gather(i):
        b = i % NBUF
        c0 = pltpu.async_copy(
            tbl_hbm.at[idx_v.at[pl.ds(i * SEQ, SPLIT_A)]],
            rows_v.at[b, pl.ds(0, SPLIT_A)], gsem.at[b])
        c1 = pltpu.async_copy(
            tbl_hbm.at[idx_v.at[pl.ds(i * SEQ + SPLIT_A, SPLIT_B)]],
            rows_v.at[b, pl.ds(SPLIT_A, SPLIT_B)], gsem.at[b])
        return c0, c1

    def start_writeback(i):
        b = i % NBUF
        return pltpu.async_copy(
            rows_v.at[b], out_hbm.at[pl.ds((base + i) * SEQ, SEQ)], wsem.at[b])

    gathers = {}
    wbs = {}
    for i in range(LOOKAHEAD):
        gathers[i] = start_gather(i)

    for i in range(CPW):
        b = i % NBUF
        g0, g1 = gathers.pop(i)
        g0.wait()
        g1.wait()

        def add_body(r, carry, _b=b):
            for k in range(DIM // 16):
                sl = pl.ds(k * 16, 16)
                rows_v[_b, r, sl] = rows_v[_b, r, sl] + pe_v[r, sl]
            return carry

        lax.fori_loop(0, SEQ, add_body, 0, unroll=2)
        wbs[i] = start_writeback(i)

        j = i + LOOKAHEAD
        if j < CPW:
            if j >= NBUF:
                wbs.pop(j - NBUF).wait()
            gathers[j] = start_gather(j)

    for i in sorted(wbs):
        wbs[i].wait()


def kernel(X, token_table):
    idx = X.astype(jnp.int32).reshape(BATCH * SEQ)
    pe = _positional_encoding(SEQ, DIM)
    tbl = _table_rowmajor(token_table).reshape(2 * VOCAB, DIM)

    mesh = plsc.VectorSubcoreMesh(core_axis_name="c", subcore_axis_name="s")
    run = functools.partial(
        pl.kernel,
        mesh=mesh,
        compiler_params=pltpu.CompilerParams(use_tc_tiling_on_sc=False),
        out_type=jax.ShapeDtypeStruct((BATCH * SEQ, DIM), jnp.float32),
        scratch_types=[
            pltpu.VMEM((CPW * SEQ,), jnp.int32),
            pltpu.VMEM((NBUF, SEQ, DIM), jnp.float32),
            pltpu.VMEM((SEQ, DIM), jnp.float32),
            pltpu.SemaphoreType.DMA((NBUF,)),
            pltpu.SemaphoreType.DMA((NBUF,)),
        ],
    )(_sc_body)
    return run(idx, tbl, pe).reshape(BATCH, SEQ, DIM)


# speedup vs baseline: 1.9500x; 1.0130x over previous
import functools

import jax
import jax.numpy as jnp
from jax import lax
from jax.experimental import pallas as pl
from jax.experimental.pallas import tpu as pltpu
from jax.experimental.pallas import tpu_sc as plsc

VOCAB = 1000000
DIM = 64
BATCH = 1024
SEQ = 200

PDIM = 128
SPLIT_A = 96
SPLIT_B = 104
NWORKERS = 32
CPW = BATCH // NWORKERS
NBUF = 4
LOOKAHEAD = 2

TBLOCK = 32768


def _positional_encoding(max_len, dim):
    pos = jnp.arange(max_len, dtype=jnp.float32)[:, None]
    div = jnp.exp(jnp.arange(0, dim, 2, dtype=jnp.float32) * (-jnp.log(10000.0) / dim))
    pe = jnp.zeros((max_len, dim), dtype=jnp.float32)
    pe = pe.at[:, 0::2].set(jnp.sin(pos * div))
    pe = pe.at[:, 1::2].set(jnp.cos(pos * div))
    return pe


def _transpose_body(tbl_t_ref, out_ref):
    out_ref[:, 0:DIM] = tbl_t_ref[...].T


def _table_rowmajor(token_table):
    grid = (VOCAB + TBLOCK - 1) // TBLOCK
    return pl.pallas_call(
        _transpose_body,
        grid=(grid,),
        in_specs=[pl.BlockSpec((DIM, TBLOCK), lambda g: (0, g))],
        out_specs=pl.BlockSpec((TBLOCK, PDIM), lambda g: (g, 0)),
        out_shape=jax.ShapeDtypeStruct((VOCAB, PDIM), jnp.float32),
    )(token_table.T)


def _sc_body(idx_hbm, tbl_hbm, pe_hbm, out_hbm, idx_v, rows_v, pe_v, gsem, wsem):
    wid = lax.axis_index("s") * 2 + lax.axis_index("c")
    base = wid * CPW

    pltpu.sync_copy(pe_hbm, pe_v)
    pltpu.sync_copy(idx_hbm.at[pl.ds(base * SEQ, CPW * SEQ)], idx_v)

    def remap_body(v, carry):
        sl = pl.ds(v * 16, 16)
        idx_v[sl] = idx_v[sl] * 2
        return carry

    lax.fori_loop(0, CPW * SEQ // 16, remap_body, 0, unroll=4)

    def start_gather(i):
        b = i % NBUF
        c0 = pltpu.async_copy(
            tbl_hbm.at[idx_v.at[pl.ds(i * SEQ, SPLIT_A)]],
            rows_v.at[b, pl.ds(0, SPLIT_A)], gsem.at[b])
        c1 = pltpu.async_copy(
            tbl_hbm.at[idx_v.at[pl.ds(i * SEQ + SPLIT_A, SPLIT_B)]],
            rows_v.at[b, pl.ds(SPLIT_A, SPLIT_B)], gsem.at[b])
        return c0, c1

    def start_writeback(i):
        b = i % NBUF
        return pltpu.async_copy(
            rows_v.at[b], out_hbm.at[pl.ds((base + i) * SEQ, SEQ)], wsem.at[b])

    gathers = {}
    wbs = {}
    for i in range(LOOKAHEAD):
        gathers[i] = start_gather(i)

    for i in range(CPW):
        b = i % NBUF
        g0, g1 = gathers.pop(i)
        g0.wait()
        g1.wait()

        def add_body(r, carry, _b=b):
            for k in range(DIM // 16):
                sl = pl.ds(k * 16, 16)
                rows_v[_b, r, sl] = rows_v[_b, r, sl] + pe_v[r, sl]
            return carry

        lax.fori_loop(0, SEQ, add_body, 0, unroll=2)
        wbs[i] = start_writeback(i)

        j = i + LOOKAHEAD
        if j < CPW:
            if j >= NBUF:
                wbs.pop(j - NBUF).wait()
            gathers[j] = start_gather(j)

    for i in sorted(wbs):
        wbs[i].wait()


def kernel(X, token_table):
    idx = X.astype(jnp.int32).reshape(BATCH * SEQ)
    pe = _positional_encoding(SEQ, DIM)
    tbl = _table_rowmajor(token_table).reshape(2 * VOCAB, DIM)

    mesh = plsc.VectorSubcoreMesh(core_axis_name="c", subcore_axis_name="s")
    run = functools.partial(
        pl.kernel,
        mesh=mesh,
        compiler_params=pltpu.CompilerParams(use_tc_tiling_on_sc=False),
        out_type=jax.ShapeDtypeStruct((BATCH * SEQ, DIM), jnp.float32),
        scratch_types=[
            pltpu.VMEM((CPW * SEQ,), jnp.int32),
            pltpu.VMEM((NBUF, SEQ, DIM), jnp.float32),
            pltpu.VMEM((SEQ, DIM), jnp.float32),
            pltpu.SemaphoreType.DMA((NBUF,)),
            pltpu.SemaphoreType.DMA((NBUF,)),
        ],
    )(_sc_body)
    return run(idx, tbl, pe).reshape(BATCH, SEQ, DIM)


# speedup vs baseline: 2.3374x; 1.1987x over previous
import functools

import jax
import jax.numpy as jnp
from jax import lax
from jax.experimental import pallas as pl
from jax.experimental.pallas import tpu as pltpu
from jax.experimental.pallas import tpu_sc as plsc

VOCAB = 1000000
DIM = 64
BATCH = 1024
SEQ = 200

PDIM = 128
SPLIT_A = 96
SPLIT_B = 104
NWORKERS = 32
CPW = BATCH // NWORKERS
NBUF = 4
LOOKAHEAD = 2

TBLOCK = 32768


def _positional_encoding(max_len, dim):
    pos = jnp.arange(max_len, dtype=jnp.float32)[:, None]
    div = jnp.exp(jnp.arange(0, dim, 2, dtype=jnp.float32) * (-jnp.log(10000.0) / dim))
    pe = jnp.zeros((max_len, dim), dtype=jnp.float32)
    pe = pe.at[:, 0::2].set(jnp.sin(pos * div))
    pe = pe.at[:, 1::2].set(jnp.cos(pos * div))
    return pe


def _transpose_body(tbl_t_ref, out_ref):
    out_ref[:, 0:DIM] = tbl_t_ref[...].T


def _table_rowmajor(token_table):
    grid = (VOCAB + TBLOCK - 1) // TBLOCK
    return pl.pallas_call(
        _transpose_body,
        grid=(grid,),
        in_specs=[pl.BlockSpec((DIM, TBLOCK), lambda g: (0, g))],
        out_specs=pl.BlockSpec((TBLOCK, PDIM), lambda g: (g, 0)),
        out_shape=jax.ShapeDtypeStruct((VOCAB, PDIM), jnp.float32),
    )(token_table.T)


def _sc_body(idx_hbm, tbl_hbm, pe_hbm, out_hbm, idx_v, rows_v, pe_v, gsem, wsem):
    wid = lax.axis_index("s") * 2 + lax.axis_index("c")
    base = wid * CPW

    pltpu.sync_copy(pe_hbm, pe_v)
    pltpu.sync_copy(idx_hbm.at[pl.ds(base * SEQ, CPW * SEQ)], idx_v)

    def remap_body(v, carry):
        sl = pl.ds(v * 16, 16)
        idx_v[sl] = idx_v[sl] * 2
        return carry

    lax.fori_loop(0, CPW * SEQ // 16, remap_body, 0, unroll=4)

    def start_gather(i):
        b = i % NBUF
        c0 = pltpu.async_copy(
            tbl_hbm.at[idx_v.at[pl.ds(i * SEQ, SPLIT_A)]],
            rows_v.at[b, pl.ds(0, SPLIT_A)], gsem.at[b])
        c1 = pltpu.async_copy(
            tbl_hbm.at[idx_v.at[pl.ds(i * SEQ + SPLIT_A, SPLIT_B)]],
            rows_v.at[b, pl.ds(SPLIT_A, SPLIT_B)], gsem.at[b])
        return c0, c1

    def start_writeback(i):
        b = i % NBUF
        return pltpu.async_copy(
            rows_v.at[b],
            out_hbm.at[pl.ds((base + i) * SEQ, SEQ), pl.ds(0, DIM)], wsem.at[b])

    gathers = {}
    wbs = {}
    for i in range(LOOKAHEAD):
        gathers[i] = start_gather(i)

    for i in range(CPW):
        b = i % NBUF
        g0, g1 = gathers.pop(i)
        g0.wait()
        g1.wait()

        def add_body(r, carry, _b=b):
            for k in range(DIM // 16):
                sl = pl.ds(k * 16, 16)
                rows_v[_b, r, sl] = rows_v[_b, r, sl] + pe_v[r, sl]
            return carry

        lax.fori_loop(0, SEQ, add_body, 0, unroll=2)
        wbs[i] = start_writeback(i)

        j = i + LOOKAHEAD
        if j < CPW:
            if j >= NBUF:
                wbs.pop(j - NBUF).wait()
            gathers[j] = start_gather(j)

    for i in sorted(wbs):
        wbs[i].wait()


def kernel(X, token_table):
    idx = X.astype(jnp.int32).reshape(BATCH * SEQ)
    pe = _positional_encoding(SEQ, DIM)
    tbl = _table_rowmajor(token_table).reshape(2 * VOCAB, DIM)

    mesh = plsc.VectorSubcoreMesh(core_axis_name="c", subcore_axis_name="s")
    run = functools.partial(
        pl.kernel,
        mesh=mesh,
        compiler_params=pltpu.CompilerParams(use_tc_tiling_on_sc=False),
        out_type=jax.ShapeDtypeStruct((BATCH * SEQ, PDIM), jnp.float32),
        scratch_types=[
            pltpu.VMEM((CPW * SEQ,), jnp.int32),
            pltpu.VMEM((NBUF, SEQ, DIM), jnp.float32),
            pltpu.VMEM((SEQ, DIM), jnp.float32),
            pltpu.SemaphoreType.DMA((NBUF,)),
            pltpu.SemaphoreType.DMA((NBUF,)),
        ],
    )(_sc_body)
    return run(idx, tbl, pe)[:, 0:DIM].reshape(BATCH, SEQ, DIM)


# speedup vs baseline: 2.4170x; 1.0341x over previous
import functools

import jax
import jax.numpy as jnp
from jax import lax
from jax.experimental import pallas as pl
from jax.experimental.pallas import tpu as pltpu
from jax.experimental.pallas import tpu_sc as plsc

VOCAB = 1000000
DIM = 64
BATCH = 1024
SEQ = 200

PDIM = 128
SPLIT_A = 96
SPLIT_B = 104
NWORKERS = 32
CPW = BATCH // NWORKERS
NBUF = 6
LOOKAHEAD = 3

TBLOCK = 32768


def _positional_encoding(max_len, dim):
    pos = jnp.arange(max_len, dtype=jnp.float32)[:, None]
    div = jnp.exp(jnp.arange(0, dim, 2, dtype=jnp.float32) * (-jnp.log(10000.0) / dim))
    pe = jnp.zeros((max_len, dim), dtype=jnp.float32)
    pe = pe.at[:, 0::2].set(jnp.sin(pos * div))
    pe = pe.at[:, 1::2].set(jnp.cos(pos * div))
    return pe


def _transpose_body(tbl_t_ref, out_ref):
    out_ref[:, 0:DIM] = tbl_t_ref[...].T


def _table_rowmajor(token_table):
    grid = (VOCAB + TBLOCK - 1) // TBLOCK
    return pl.pallas_call(
        _transpose_body,
        grid=(grid,),
        in_specs=[pl.BlockSpec((DIM, TBLOCK), lambda g: (0, g))],
        out_specs=pl.BlockSpec((TBLOCK, PDIM), lambda g: (g, 0)),
        out_shape=jax.ShapeDtypeStruct((VOCAB, PDIM), jnp.float32),
    )(token_table.T)


def _sc_body(idx_hbm, tbl_hbm, pe_hbm, out_hbm, idx_v, rows_v, pe_v, gsem, wsem):
    wid = lax.axis_index("s") * 2 + lax.axis_index("c")
    base = wid * CPW

    pltpu.sync_copy(pe_hbm, pe_v)
    pltpu.sync_copy(idx_hbm.at[pl.ds(base * SEQ, CPW * SEQ)], idx_v)

    def remap_body(v, carry):
        sl = pl.ds(v * 16, 16)
        idx_v[sl] = idx_v[sl] * 2
        return carry

    lax.fori_loop(0, CPW * SEQ // 16, remap_body, 0, unroll=4)

    def start_gather(i):
        b = i % NBUF
        c0 = pltpu.async_copy(
            tbl_hbm.at[idx_v.at[pl.ds(i * SEQ, SPLIT_A)]],
            rows_v.at[b, pl.ds(0, SPLIT_A)], gsem.at[b])
        c1 = pltpu.async_copy(
            tbl_hbm.at[idx_v.at[pl.ds(i * SEQ + SPLIT_A, SPLIT_B)]],
            rows_v.at[b, pl.ds(SPLIT_A, SPLIT_B)], gsem.at[b])
        return c0, c1

    def start_writeback(i):
        b = i % NBUF
        return pltpu.async_copy(
            rows_v.at[b],
            out_hbm.at[pl.ds((base + i) * SEQ, SEQ), pl.ds(0, DIM)], wsem.at[b])

    gathers = {}
    wbs = {}
    for i in range(LOOKAHEAD):
        gathers[i] = start_gather(i)

    for i in range(CPW):
        b = i % NBUF
        g0, g1 = gathers.pop(i)
        g0.wait()
        g1.wait()

        def add_body(r, carry, _b=b):
            for k in range(DIM // 16):
                sl = pl.ds(k * 16, 16)
                rows_v[_b, r, sl] = rows_v[_b, r, sl] + pe_v[r, sl]
            return carry

        lax.fori_loop(0, SEQ, add_body, 0, unroll=2)
        wbs[i] = start_writeback(i)

        j = i + LOOKAHEAD
        if j < CPW:
            if j >= NBUF:
                wbs.pop(j - NBUF).wait()
            gathers[j] = start_gather(j)

    for i in sorted(wbs):
        wbs[i].wait()


def kernel(X, token_table):
    idx = X.astype(jnp.int32).reshape(BATCH * SEQ)
    pe = _positional_encoding(SEQ, DIM)
    tbl = _table_rowmajor(token_table).reshape(2 * VOCAB, DIM)

    mesh = plsc.VectorSubcoreMesh(core_axis_name="c", subcore_axis_name="s")
    run = functools.partial(
        pl.kernel,
        mesh=mesh,
        compiler_params=pltpu.CompilerParams(use_tc_tiling_on_sc=False),
        out_type=jax.ShapeDtypeStruct((BATCH * SEQ, PDIM), jnp.float32),
        scratch_types=[
            pltpu.VMEM((CPW * SEQ,), jnp.int32),
            pltpu.VMEM((NBUF, SEQ, DIM), jnp.float32),
            pltpu.VMEM((SEQ, DIM), jnp.float32),
            pltpu.SemaphoreType.DMA((NBUF,)),
            pltpu.SemaphoreType.DMA((NBUF,)),
        ],
    )(_sc_body)
    return run(idx, tbl, pe)[:, 0:DIM].reshape(BATCH, SEQ, DIM)
